# Initial kernel scaffold; baseline (speedup 1.0000x reference)
import jax
import jax.numpy as jnp
from jax.experimental import pallas as pl


def kernel(x, edge_index, W1l, b1, W1r, W2l, b2, W2r, Wal, ba, War, wm, bm, wv, bv):
    raise NotImplementedError("write your pallas kernel here")


# speedup vs baseline: 10.2691x; 10.2691x over previous
import functools

import jax
import jax.numpy as jnp
from jax import lax
from jax.experimental import pallas as pl
from jax.experimental.pallas import tpu as pltpu
from jax.experimental.pallas import tpu_sc as plsc

N = 10000
D = 128
NPAD = 10240
E = 320000
NW = 32
EW = 10240
EP = NW * EW
C = 128
CH = EW // C
ZR = 128
RPT = NPAD // 16


def _sc_aggregate(xp, srcp, dstp):
    @functools.partial(
        pl.kernel,
        out_type=[
            jax.ShapeDtypeStruct((2, NPAD, D), jnp.float32),
            jax.ShapeDtypeStruct((NW, NPAD), jnp.float32),
        ],
        mesh=plsc.VectorSubcoreMesh(core_axis_name="c", subcore_axis_name="s"),
        compiler_params=pltpu.CompilerParams(needs_layout_passes=False),
        scratch_types=[
            pltpu.VMEM_SHARED((NPAD, D), jnp.float32),
            pltpu.VMEM((C,), jnp.int32),
            pltpu.VMEM((C,), jnp.int32),
            pltpu.VMEM((C, D), jnp.float32),
            pltpu.VMEM((NPAD,), jnp.float32),
            pltpu.VMEM((ZR, D), jnp.float32),
            pltpu.SemaphoreType.DMA,
        ],
    )
    def k(x_hbm, src_hbm, dst_hbm, part_hbm, cnt_hbm,
          acc_sh, sidx_v, didx_v, rows_v, cnt_v, zbuf, sem):
        cid = lax.axis_index("c")
        sid = lax.axis_index("s")
        wid = cid * 16 + sid
        zeros16 = jnp.zeros((16,), jnp.float32)

        def zrow(r, carry):
            for i in range(D // 16):
                zbuf[r, pl.ds(i * 16, 16)] = zeros16
            return carry

        lax.fori_loop(0, ZR, zrow, 0)

        def zcnt(j, carry):
            cnt_v[pl.ds(j * 16, 16)] = zeros16
            return carry

        lax.fori_loop(0, NPAD // 16, zcnt, 0)

        for kk in range(RPT // ZR):
            pltpu.sync_copy(zbuf, acc_sh.at[pl.ds(sid * RPT + kk * ZR, ZR)])
        plsc.subcore_barrier()

        ones16 = jnp.ones((16,), jnp.float32)

        def chunk(g, carry):
            base = wid * EW + g * C
            pltpu.sync_copy(src_hbm.at[pl.ds(base, C)], sidx_v)
            pltpu.sync_copy(dst_hbm.at[pl.ds(base, C)], didx_v)
            pltpu.async_copy(x_hbm.at[sidx_v], rows_v, sem).wait()
            pltpu.sync_copy(rows_v, acc_sh.at[didx_v], add=True)
            for t in range(C // 16):
                dv = didx_v[pl.ds(t * 16, 16)]
                plsc.addupdate_scatter(cnt_v, [dv], ones16)
            return carry

        lax.fori_loop(0, CH, chunk, 0)
        plsc.subcore_barrier()

        for kk in range(RPT // ZR):
            r0 = sid * RPT + kk * ZR
            pltpu.sync_copy(acc_sh.at[pl.ds(r0, ZR)],
                            part_hbm.at[cid, pl.ds(r0, ZR)])
        pltpu.sync_copy(cnt_v, cnt_hbm.at[wid])

    return k(xp, srcp, dstp)


def _tc_layer(part, cntp, xp, w1lT, b1r, w1rT, w2):
    def body(part_ref, cnt_ref, x_ref, wl_ref, b1_ref, wr_ref, w2_ref,
             pq_ref, cntc_ref):
        s = part_ref[0] + part_ref[1]
        cnt = jnp.sum(cnt_ref[...], axis=0)
        cntc = jnp.maximum(cnt, 1.0)
        aggr = s / cntc[:, None]
        h = jnp.maximum(
            jnp.dot(aggr, wl_ref[...], preferred_element_type=jnp.float32)
            + b1_ref[...]
            + jnp.dot(x_ref[...], wr_ref[...], preferred_element_type=jnp.float32),
            0.0)
        pq_ref[...] = jnp.dot(h, w2_ref[...], preferred_element_type=jnp.float32)
        cntc_ref[...] = cntc

    B = 512
    grid = NPAD // B
    return pl.pallas_call(
        body,
        grid=(grid,),
        in_specs=[
            pl.BlockSpec((2, B, D), lambda i: (0, i, 0)),
            pl.BlockSpec((NW, B), lambda i: (0, i)),
            pl.BlockSpec((B, D), lambda i: (i, 0)),
            pl.BlockSpec((D, D), lambda i: (0, 0)),
            pl.BlockSpec((1, D), lambda i: (0, 0)),
            pl.BlockSpec((D, D), lambda i: (0, 0)),
            pl.BlockSpec((D, 2), lambda i: (0, 0)),
        ],
        out_specs=[
            pl.BlockSpec((B, 2), lambda i: (i, 0)),
            pl.BlockSpec((B,), lambda i: (i,)),
        ],
        out_shape=[
            jax.ShapeDtypeStruct((NPAD, 2), jnp.float32),
            jax.ShapeDtypeStruct((NPAD,), jnp.float32),
        ],
    )(part, cntp, xp, w1lT, b1r, w1rT, w2)


def _sc_scalar_aggregate(p, srcp, dstp):
    @functools.partial(
        pl.kernel,
        out_type=jax.ShapeDtypeStruct((NW, NPAD), jnp.float32),
        mesh=plsc.VectorSubcoreMesh(core_axis_name="c", subcore_axis_name="s"),
        compiler_params=pltpu.CompilerParams(needs_layout_passes=False),
        scratch_types=[
            pltpu.VMEM((NPAD,), jnp.float32),
            pltpu.VMEM((NPAD,), jnp.float32),
            pltpu.VMEM((EW,), jnp.int32),
            pltpu.VMEM((EW,), jnp.int32),
        ],
    )
    def k(p_hbm, src_hbm, dst_hbm, out_hbm, p_v, acc_v, sidx_v, didx_v):
        cid = lax.axis_index("c")
        sid = lax.axis_index("s")
        wid = cid * 16 + sid
        pltpu.sync_copy(p_hbm, p_v)
        pltpu.sync_copy(src_hbm.at[pl.ds(wid * EW, EW)], sidx_v)
        pltpu.sync_copy(dst_hbm.at[pl.ds(wid * EW, EW)], didx_v)
        zeros16 = jnp.zeros((16,), jnp.float32)

        def zacc(j, carry):
            acc_v[pl.ds(j * 16, 16)] = zeros16
            return carry

        lax.fori_loop(0, NPAD // 16, zacc, 0)

        def step(j, carry):
            si = sidx_v[pl.ds(j * 16, 16)]
            dv = didx_v[pl.ds(j * 16, 16)]
            vals = plsc.load_gather(p_v, [si])
            plsc.addupdate_scatter(acc_v, [dv], vals)
            return carry

        lax.fori_loop(0, EW // 16, step, 0)
        pltpu.sync_copy(acc_v, out_hbm.at[wid])

    return k(p, srcp, dstp)


def _tc_final(pacc_r, cntc_r, q_r, noise_r, scal):
    def body(sc_ref, pacc_ref, cntc_ref, q_ref, noise_ref, out_ref):
        a = jnp.sum(pacc_ref[...], axis=0)
        h2 = a / cntc_ref[...] + sc_ref[0] + q_ref[...]
        out_ref[...] = (h2 * sc_ref[1] + sc_ref[2]
                        + noise_ref[...] * jnp.exp(h2 * sc_ref[3] + sc_ref[4]))

    R = NPAD // 128
    return pl.pallas_call(
        body,
        in_specs=[
            pl.BlockSpec(memory_space=pltpu.SMEM),
            pl.BlockSpec((NW, R, 128), lambda: (0, 0, 0)),
            pl.BlockSpec((R, 128), lambda: (0, 0)),
            pl.BlockSpec((R, 128), lambda: (0, 0)),
            pl.BlockSpec((R, 128), lambda: (0, 0)),
        ],
        out_specs=pl.BlockSpec((R, 128), lambda: (0, 0)),
        out_shape=jax.ShapeDtypeStruct((R, 128), jnp.float32),
    )(scal, pacc_r, cntc_r, q_r, noise_r)


def kernel(x, edge_index, W1l, b1, W1r, W2l, b2, W2r, Wal, ba, War, wm, bm, wv, bv):
    src = edge_index[0].astype(jnp.int32)
    dst = edge_index[1].astype(jnp.int32)
    pad_e = EP - E
    srcp = jnp.concatenate([src, jnp.zeros((pad_e,), jnp.int32)])
    dstp = jnp.concatenate([dst, jnp.full((pad_e,), N, jnp.int32)])
    xp = jnp.pad(x, ((0, NPAD - N), (0, 0)))

    part, cntp = _sc_aggregate(xp, srcp, dstp)

    w2 = jnp.stack([W2l[0], W2r[0]], axis=1)
    pq, cntc = _tc_layer(part, cntp, xp, W1l.T, b1.reshape(1, D), W1r.T, w2)

    p = pq[:, 0]
    pacc = _sc_scalar_aggregate(p, srcp, dstp)

    R = NPAD // 128
    noise = jax.random.normal(jax.random.key(42), (N, 1), jnp.float32)
    noise_r = jnp.pad(noise[:, 0], (0, NPAD - N)).reshape(R, 128)
    q_r = pq[:, 1].reshape(R, 128)
    cntc_r = cntc.reshape(R, 128)
    pacc_r = pacc.reshape(NW, R, 128)
    scal = jnp.concatenate([b2, wm.ravel(), bm, wv.ravel(), bv])

    out_r = _tc_final(pacc_r, cntc_r, q_r, noise_r, scal)
    return out_r.reshape(NPAD)[:N][:, None]
